# baseline (speedup 1.0000x reference)
import functools

import jax
import jax.numpy as jnp
from jax import lax
from jax.experimental import pallas as pl
from jax.experimental.pallas import tpu as pltpu
from jax.experimental.pallas import tpu_sc as plsc

B = 16384
DIMS = (32, 16, 16, 8, 4, 4, 4, 4, 16)
PDIMS = (32, 16, 16, 16, 16, 16, 16, 16, 16)
NC, NS = 2, 16
NW = NC * NS
BPW = B // NW

_F32 = jnp.float32


def _sc_gather_kernel():
    mesh = plsc.VectorSubcoreMesh(core_axis_name="c", subcore_axis_name="s")
    out_type = tuple(jax.ShapeDtypeStruct((B, d), _F32) for d in PDIMS)
    scratch = (
        [pltpu.VMEM((BPW,), jnp.int32)]
        + [pltpu.VMEM((BPW,), jnp.int32)] * 8
        + [pltpu.VMEM((BPW, d), _F32) for d in PDIMS]
        + [pltpu.SemaphoreType.DMA]
    )

    @functools.partial(
        pl.kernel,
        out_type=out_type,
        mesh=mesh,
        scratch_types=scratch,
        compiler_params=pltpu.CompilerParams(use_tc_tiling_on_sc=False),
    )
    def body(x, d_company, d_area, d_region, d_ws, d_emp, d_we, d_cur, d_name,
             vac_t, comp_t, area_t, reg_t, emp_t, ws_t, we_t, cur_t, name_t,
             o_vac, o_comp, o_area, o_reg, o_ws, o_emp, o_we, o_cur, o_name,
             xv, i_comp, i_area, i_reg, i_ws, i_emp, i_we, i_cur, i_name,
             r_vac, r_comp, r_area, r_reg, r_ws, r_emp, r_we, r_cur, r_name,
             sem):
        wid = lax.axis_index("s") * NC + lax.axis_index("c")
        base = wid * BPW
        pltpu.sync_copy(x.at[pl.ds(base, BPW)], xv)

        nchunk = BPW // 128

        def gather(tbl, idx, dst):
            cps = []
            for ck in range(nchunk):
                sl = pl.ds(ck * 128, 128)
                cps.append(pltpu.async_copy(tbl.at[idx.at[sl]], dst.at[sl], sem))
            return cps

        lvl1 = gather(vac_t, xv, r_vac)
        attr_tables = (d_company, d_area, d_region, d_ws, d_emp, d_we, d_cur, d_name)
        attr_idx = (i_comp, i_area, i_reg, i_ws, i_emp, i_we, i_cur, i_name)
        for tbl, dst in zip(attr_tables, attr_idx):
            lvl1 += gather(tbl, xv, dst)
        for cp in lvl1:
            cp.wait()

        emb_tables = (comp_t, area_t, reg_t, ws_t, emp_t, we_t, cur_t, name_t)
        emb_idx = (i_comp, i_area, i_reg, i_ws, i_emp, i_we, i_cur, i_name)
        emb_rows = (r_comp, r_area, r_reg, r_ws, r_emp, r_we, r_cur, r_name)
        lvl2 = []
        for t, i, r in zip(emb_tables, emb_idx, emb_rows):
            lvl2 += gather(t, i, r)
        for cp in lvl2:
            cp.wait()

        outs = (o_vac, o_comp, o_area, o_reg, o_ws, o_emp, o_we, o_cur, o_name)
        rows = (r_vac, r_comp, r_area, r_reg, r_ws, r_emp, r_we, r_cur, r_name)
        for r, o in zip(rows, outs):
            pltpu.sync_copy(r, o.at[pl.ds(base, BPW)])

    return body


_GATHER = _sc_gather_kernel()

_BLK = 2048


def _mm_body(f0, f1, f2, f3, f4, f5, f6, f7, f8,
             w0, w1, w2, w3, w4, w5, w6, w7, w8, b, out):
    acc = jnp.dot(f0[...], w0[...], preferred_element_type=_F32)
    for f, w in ((f1, w1), (f2, w2), (f3, w3), (f4, w4),
                 (f5, w5), (f6, w6), (f7, w7), (f8, w8)):
        acc = acc + jnp.dot(f[...], w[...], preferred_element_type=_F32)
    out[...] = acc + b[...]


def _projection(feats, ws, b):
    n_blk = B // _BLK
    in_specs = (
        [pl.BlockSpec((_BLK, d), lambda i: (i, 0)) for d in PDIMS]
        + [pl.BlockSpec((d, 64), lambda i: (0, 0)) for d in PDIMS]
        + [pl.BlockSpec((1, 64), lambda i: (0, 0))]
    )
    return pl.pallas_call(
        _mm_body,
        grid=(n_blk,),
        in_specs=in_specs,
        out_specs=pl.BlockSpec((_BLK, 64), lambda i: (i, 0)),
        out_shape=jax.ShapeDtypeStruct((B, 64), _F32),
    )(*feats, *ws, b.reshape(1, 64))


def kernel(x, d_company, d_area, d_region, d_ws, d_emp, d_we, d_cur, d_name,
           vac_t, comp_t, area_t, reg_t, emp_t, ws_t, we_t, cur_t, name_t, W, b):
    def pad16(t):
        return jnp.pad(t, ((0, 0), (0, 16 - t.shape[1])))

    tables = (vac_t, comp_t, area_t, pad16(reg_t), pad16(emp_t), pad16(ws_t),
              pad16(we_t), pad16(cur_t), name_t)
    feats = _GATHER(x.astype(jnp.int32),
                    d_company, d_area, d_region, d_ws, d_emp, d_we, d_cur, d_name,
                    *tables)
    offs, ws = 0, []
    for d, pd in zip(DIMS, PDIMS):
        w = W[offs:offs + d, :]
        if pd != d:
            w = jnp.pad(w, ((0, pd - d), (0, 0)))
        ws.append(w)
        offs += d
    return _projection(feats, ws, b)


# speedup vs baseline: 1.0048x; 1.0048x over previous
import functools

import jax
import jax.numpy as jnp
from jax import lax
from jax.experimental import pallas as pl
from jax.experimental.pallas import tpu as pltpu
from jax.experimental.pallas import tpu_sc as plsc

B = 16384
N_VAC = 2734130
NP_VAC = (N_VAC * 32) // 128
DIMS = (32, 16, 16, 8, 4, 4, 4, 4, 16)
ADIMS = (16, 16, 16, 16, 16, 16, 16, 16)
NC, NS = 2, 16
NW = NC * NS
BPW = B // NW

_F32 = jnp.float32


def _sc_gather_kernel():
    mesh = plsc.VectorSubcoreMesh(core_axis_name="c", subcore_axis_name="s")
    out_type = (jax.ShapeDtypeStruct((B, 128), _F32),) + tuple(
        jax.ShapeDtypeStruct((B, d), _F32) for d in ADIMS)
    scratch = (
        [pltpu.VMEM((BPW,), jnp.int32)]
        + [pltpu.VMEM((BPW,), jnp.int32)]
        + [pltpu.VMEM((BPW,), jnp.int32)] * 8
        + [pltpu.VMEM((128, 128), _F32)] * 2
        + [pltpu.VMEM((BPW, d), _F32) for d in ADIMS]
        + [pltpu.SemaphoreType.DMA] * 3
    )

    @functools.partial(
        pl.kernel,
        out_type=out_type,
        mesh=mesh,
        scratch_types=scratch,
        compiler_params=pltpu.CompilerParams(use_tc_tiling_on_sc=False),
    )
    def body(x, xp, d_company, d_area, d_region, d_ws, d_emp, d_we, d_cur, d_name,
             vac_p, comp_t, area_t, reg_t, emp_t, ws_t, we_t, cur_t, name_t,
             o_vac, o_comp, o_area, o_reg, o_ws, o_emp, o_we, o_cur, o_name,
             xv, xpv, i_comp, i_area, i_reg, i_ws, i_emp, i_we, i_cur, i_name,
             vb0, vb1,
             r_comp, r_area, r_reg, r_ws, r_emp, r_we, r_cur, r_name,
             sem, sem_v0, sem_v1):
        wid = lax.axis_index("s") * NC + lax.axis_index("c")
        base = wid * BPW
        pltpu.sync_copy(x.at[pl.ds(base, BPW)], xv)
        pltpu.sync_copy(xp.at[pl.ds(base, BPW)], xpv)

        nchunk = BPW // 128

        def gather(tbl, idx, dst):
            cps = []
            for ck in range(nchunk):
                sl = pl.ds(ck * 128, 128)
                cps.append(pltpu.async_copy(tbl.at[idx.at[sl]], dst.at[sl], sem))
            return cps

        lvl1 = []
        attr_tables = (d_company, d_area, d_region, d_ws, d_emp, d_we, d_cur, d_name)
        attr_idx = (i_comp, i_area, i_reg, i_ws, i_emp, i_we, i_cur, i_name)
        for tbl, dst in zip(attr_tables, attr_idx):
            lvl1 += gather(tbl, xv, dst)

        vbufs = (vb0, vb1)
        vsems = (sem_v0, sem_v1)
        pend = []

        def drain_one():
            cp0, k0 = pend.pop(0)
            cp0.wait()
            pltpu.sync_copy(vbufs[k0 % 2], o_vac.at[pl.ds(base + k0 * 128, 128)])

        for ck in range(nchunk):
            if len(pend) == 2:
                drain_one()
            cp = pltpu.async_copy(
                vac_p.at[xpv.at[pl.ds(ck * 128, 128)]], vbufs[ck % 2],
                vsems[ck % 2])
            pend.append((cp, ck))

        for cp in lvl1:
            cp.wait()

        emb_tables = (comp_t, area_t, reg_t, ws_t, emp_t, we_t, cur_t, name_t)
        emb_idx = (i_comp, i_area, i_reg, i_ws, i_emp, i_we, i_cur, i_name)
        emb_rows = (r_comp, r_area, r_reg, r_ws, r_emp, r_we, r_cur, r_name)
        lvl2 = []
        for t, i, r in zip(emb_tables, emb_idx, emb_rows):
            lvl2 += gather(t, i, r)

        while pend:
            drain_one()
        for cp in lvl2:
            cp.wait()

        outs = (o_comp, o_area, o_reg, o_ws, o_emp, o_we, o_cur, o_name)
        for r, o in zip(emb_rows, outs):
            pltpu.sync_copy(r, o.at[pl.ds(base, BPW)])

    return body


_GATHER = _sc_gather_kernel()

_BLK = 2048


def _mm_body(xr, fv, f1, f2, f3, f4, f5, f6, f7, f8,
             wv, tw, w1, w2, w3, w4, w5, w6, w7, w8, b, out):
    xc = xr[...]
    cols = lax.broadcasted_iota(jnp.int32, (_BLK, 128), 1)
    mask = (cols // 32 == xc % 4) & (xc < (NP_VAC * 4))
    acc = jnp.dot(fv[...] * mask.astype(_F32), wv[...],
                  preferred_element_type=_F32)
    for f, w in ((f1, w1), (f2, w2), (f3, w3), (f4, w4),
                 (f5, w5), (f6, w6), (f7, w7), (f8, w8)):
        acc = acc + jnp.dot(f[...], w[...], preferred_element_type=_F32)
    t0 = (xc == NP_VAC * 4).astype(_F32) * tw[0:1, :]
    t1 = (xc == NP_VAC * 4 + 1).astype(_F32) * tw[1:2, :]
    out[...] = acc + t0 + t1 + b[...]


def _projection(x, feats, wv, tail_w, ws, b):
    n_blk = B // _BLK
    in_specs = (
        [pl.BlockSpec((_BLK, 1), lambda i: (i, 0)),
         pl.BlockSpec((_BLK, 128), lambda i: (i, 0))]
        + [pl.BlockSpec((_BLK, d), lambda i: (i, 0)) for d in ADIMS]
        + [pl.BlockSpec((128, 64), lambda i: (0, 0)),
           pl.BlockSpec((2, 64), lambda i: (0, 0))]
        + [pl.BlockSpec((d, 64), lambda i: (0, 0)) for d in ADIMS]
        + [pl.BlockSpec((1, 64), lambda i: (0, 0))]
    )
    return pl.pallas_call(
        _mm_body,
        grid=(n_blk,),
        in_specs=in_specs,
        out_specs=pl.BlockSpec((_BLK, 64), lambda i: (i, 0)),
        out_shape=jax.ShapeDtypeStruct((B, 64), _F32),
    )(x.reshape(B, 1), *feats, wv, tail_w, *ws, b.reshape(1, 64))


def kernel(x, d_company, d_area, d_region, d_ws, d_emp, d_we, d_cur, d_name,
           vac_t, comp_t, area_t, reg_t, emp_t, ws_t, we_t, cur_t, name_t, W, b):
    x = x.astype(jnp.int32)
    vac_p = vac_t.reshape(-1)[: NP_VAC * 128].reshape(NP_VAC, 128)
    xp = jnp.minimum(x // 4, NP_VAC - 1)

    def pad16(t):
        return jnp.pad(t, ((0, 0), (0, 16 - t.shape[1])))

    tables = (comp_t, area_t, pad16(reg_t), pad16(emp_t), pad16(ws_t),
              pad16(we_t), pad16(cur_t), name_t)
    feats = _GATHER(x, xp,
                    d_company, d_area, d_region, d_ws, d_emp, d_we, d_cur, d_name,
                    vac_p, *tables)
    w_vac = W[:32, :]
    wv = jnp.tile(w_vac, (4, 1))
    tail_w = vac_t[N_VAC - 2:, :] @ w_vac
    offs, ws = 0, []
    for d, pd in zip(DIMS[1:], ADIMS):
        offs_d = 32 + offs
        w = W[offs_d:offs_d + d, :]
        if pd != d:
            w = jnp.pad(w, ((0, pd - d), (0, 0)))
        ws.append(w)
        offs += d
    return _projection(x, feats, wv, tail_w, ws, b)


# speedup vs baseline: 1.0112x; 1.0064x over previous
import functools

import jax
import jax.numpy as jnp
from jax import lax
from jax.experimental import pallas as pl
from jax.experimental.pallas import tpu as pltpu
from jax.experimental.pallas import tpu_sc as plsc

B = 16384
N_VAC = 2734130
NP_VAC = (N_VAC * 32) // 128
DIMS = (32, 16, 16, 8, 4, 4, 4, 4, 16)
ADIMS = (16, 16, 16, 16, 16, 16, 16, 16)
NC, NS = 2, 16
NW = NC * NS
BPW = B // NW
NCHUNK = BPW // 128

_F32 = jnp.float32


def _sc_vac_kernel():
    mesh = plsc.VectorSubcoreMesh(core_axis_name="c", subcore_axis_name="s")

    @functools.partial(
        pl.kernel,
        out_type=jax.ShapeDtypeStruct((B, 128), _F32),
        mesh=mesh,
        scratch_types=(
            pltpu.VMEM((NCHUNK, 128), jnp.int32),
            pltpu.VMEM((128, 128), _F32),
            pltpu.VMEM((128, 128), _F32),
            pltpu.SemaphoreType.DMA,
            pltpu.SemaphoreType.DMA,
        ),
        compiler_params=pltpu.CompilerParams(use_tc_tiling_on_sc=True),
    )
    def body(xp, vac_p, o_vac, xpv, vb0, vb1, s0, s1):
        wid = lax.axis_index("s") * NC + lax.axis_index("c")
        base = wid * BPW
        pltpu.sync_copy(xp.at[pl.ds(wid * NCHUNK, NCHUNK)], xpv)

        vbufs = (vb0, vb1)
        vsems = (s0, s1)
        pend = []

        def drain_one():
            cp0, k0 = pend.pop(0)
            cp0.wait()
            pltpu.sync_copy(vbufs[k0 % 2], o_vac.at[pl.ds(base + k0 * 128, 128)])

        for ck in range(NCHUNK):
            if len(pend) == 2:
                drain_one()
            cp = pltpu.async_copy(vac_p.at[xpv.at[ck]], vbufs[ck % 2],
                                  vsems[ck % 2])
            pend.append((cp, ck))
        while pend:
            drain_one()

    return body


def _sc_attr_kernel():
    mesh = plsc.VectorSubcoreMesh(core_axis_name="c", subcore_axis_name="s")
    out_type = tuple(jax.ShapeDtypeStruct((B, d), _F32) for d in ADIMS)
    scratch = (
        [pltpu.VMEM((BPW,), jnp.int32)]
        + [pltpu.VMEM((BPW,), jnp.int32)] * 8
        + [pltpu.VMEM((BPW, d), _F32) for d in ADIMS]
        + [pltpu.SemaphoreType.DMA]
    )

    @functools.partial(
        pl.kernel,
        out_type=out_type,
        mesh=mesh,
        scratch_types=scratch,
        compiler_params=pltpu.CompilerParams(use_tc_tiling_on_sc=False),
    )
    def body(x, d_company, d_area, d_region, d_ws, d_emp, d_we, d_cur, d_name,
             comp_t, area_t, reg_t, emp_t, ws_t, we_t, cur_t, name_t,
             o_comp, o_area, o_reg, o_ws, o_emp, o_we, o_cur, o_name,
             xv, i_comp, i_area, i_reg, i_ws, i_emp, i_we, i_cur, i_name,
             r_comp, r_area, r_reg, r_ws, r_emp, r_we, r_cur, r_name,
             sem):
        wid = lax.axis_index("s") * NC + lax.axis_index("c")
        base = wid * BPW
        pltpu.sync_copy(x.at[pl.ds(base, BPW)], xv)

        def gather(tbl, idx, dst):
            cps = []
            for ck in range(NCHUNK):
                sl = pl.ds(ck * 128, 128)
                cps.append(pltpu.async_copy(tbl.at[idx.at[sl]], dst.at[sl], sem))
            return cps

        lvl1 = []
        attr_tables = (d_company, d_area, d_region, d_ws, d_emp, d_we, d_cur, d_name)
        attr_idx = (i_comp, i_area, i_reg, i_ws, i_emp, i_we, i_cur, i_name)
        for tbl, dst in zip(attr_tables, attr_idx):
            lvl1 += gather(tbl, xv, dst)
        for cp in lvl1:
            cp.wait()

        emb_tables = (comp_t, area_t, reg_t, ws_t, emp_t, we_t, cur_t, name_t)
        emb_idx = (i_comp, i_area, i_reg, i_ws, i_emp, i_we, i_cur, i_name)
        emb_rows = (r_comp, r_area, r_reg, r_ws, r_emp, r_we, r_cur, r_name)
        lvl2 = []
        for t, i, r in zip(emb_tables, emb_idx, emb_rows):
            lvl2 += gather(t, i, r)
        for cp in lvl2:
            cp.wait()

        outs = (o_comp, o_area, o_reg, o_ws, o_emp, o_we, o_cur, o_name)
        for r, o in zip(emb_rows, outs):
            pltpu.sync_copy(r, o.at[pl.ds(base, BPW)])

    return body


_VAC_GATHER = _sc_vac_kernel()
_ATTR_GATHER = _sc_attr_kernel()

_BLK = 2048


def _mm_body(xr, fv, f1, f2, f3, f4, f5, f6, f7, f8,
             wv, tw, w1, w2, w3, w4, w5, w6, w7, w8, b, out):
    xc = xr[...]
    cols = lax.broadcasted_iota(jnp.int32, (_BLK, 128), 1)
    mask = (cols // 32 == xc % 4) & (xc < (NP_VAC * 4))
    acc = jnp.dot(fv[...] * mask.astype(_F32), wv[...],
                  preferred_element_type=_F32)
    for f, w in ((f1, w1), (f2, w2), (f3, w3), (f4, w4),
                 (f5, w5), (f6, w6), (f7, w7), (f8, w8)):
        acc = acc + jnp.dot(f[...], w[...], preferred_element_type=_F32)
    t0 = (xc == NP_VAC * 4).astype(_F32) * tw[0:1, :]
    t1 = (xc == NP_VAC * 4 + 1).astype(_F32) * tw[1:2, :]
    out[...] = acc + t0 + t1 + b[...]


def _projection(x, feats, wv, tail_w, ws, b):
    n_blk = B // _BLK
    in_specs = (
        [pl.BlockSpec((_BLK, 1), lambda i: (i, 0)),
         pl.BlockSpec((_BLK, 128), lambda i: (i, 0))]
        + [pl.BlockSpec((_BLK, d), lambda i: (i, 0)) for d in ADIMS]
        + [pl.BlockSpec((128, 64), lambda i: (0, 0)),
           pl.BlockSpec((2, 64), lambda i: (0, 0))]
        + [pl.BlockSpec((d, 64), lambda i: (0, 0)) for d in ADIMS]
        + [pl.BlockSpec((1, 64), lambda i: (0, 0))]
    )
    return pl.pallas_call(
        _mm_body,
        grid=(n_blk,),
        in_specs=in_specs,
        out_specs=pl.BlockSpec((_BLK, 64), lambda i: (i, 0)),
        out_shape=jax.ShapeDtypeStruct((B, 64), _F32),
    )(x.reshape(B, 1), *feats, wv, tail_w, *ws, b.reshape(1, 64))


def kernel(x, d_company, d_area, d_region, d_ws, d_emp, d_we, d_cur, d_name,
           vac_t, comp_t, area_t, reg_t, emp_t, ws_t, we_t, cur_t, name_t, W, b):
    x = x.astype(jnp.int32)
    vac_p = vac_t.reshape(-1)[: NP_VAC * 128].reshape(NP_VAC, 128)
    xp = jnp.minimum(x // 4, NP_VAC - 1).reshape(B // 128, 128)
    f_vac = _VAC_GATHER(xp, vac_p)

    def pad16(t):
        return jnp.pad(t, ((0, 0), (0, 16 - t.shape[1])))

    tables = (comp_t, area_t, pad16(reg_t), pad16(emp_t), pad16(ws_t),
              pad16(we_t), pad16(cur_t), name_t)
    feats = _ATTR_GATHER(x, d_company, d_area, d_region, d_ws, d_emp, d_we,
                         d_cur, d_name, *tables)

    w_vac = W[:32, :]
    wv = jnp.tile(w_vac, (4, 1))
    tail_w = vac_t[N_VAC - 2:, :] @ w_vac
    offs, ws = 32, []
    for d, pd in zip(DIMS[1:], ADIMS):
        w = W[offs:offs + d, :]
        if pd != d:
            w = jnp.pad(w, ((0, pd - d), (0, 0)))
        ws.append(w)
        offs += d
    return _projection(x, (f_vac,) + feats, wv, tail_w, ws, b)
